# baseline (speedup 1.0000x reference)
import functools

import jax
import jax.numpy as jnp
from jax import lax
from jax.experimental import pallas as pl
from jax.experimental.pallas import tpu as pltpu
from jax.experimental.pallas import tpu_sc as plsc

_BETA = 0.25
_N_EMB = 8192
_DIM = 256
_B = 16384

_BM = 256
_BNB = 2048
_NB = _N_EMB // _BNB
_LANES = 128
_NCHB = _BNB // _LANES


def _argmin_body(x_ref, w_ref, ind_ref, loss_ref, swb_ref):
    m = pl.program_id(0)
    m_last = pl.num_programs(0) - 1

    @pl.when(m == 0)
    def _():
        w = w_ref[...]
        swb_ref[...] = (0.5 * jnp.sum(w * w, axis=1)).reshape(
            _N_EMB // _LANES, _LANES)

    x = x_ref[...]

    rv = jnp.full((_BM, _LANES), jnp.inf, jnp.float32)
    rc = jnp.zeros((_BM, _LANES), jnp.int32)
    for nb in range(_NB):
        dotm = lax.dot_general(
            x, w_ref[nb * _BNB:(nb + 1) * _BNB, :],
            (((1,), (1,)), ((), ())),
            preferred_element_type=jnp.float32,
        )
        for c in range(_NCHB):
            g = nb * _NCHB + c
            d = swb_ref[pl.ds(g, 1), :] - dotm[:, c * _LANES:(c + 1) * _LANES]
            bet = d < rv
            rv = jnp.where(bet, d, rv)
            rc = jnp.where(bet, g, rc)

    mn = jnp.min(rv, axis=1, keepdims=True)
    lane = lax.broadcasted_iota(jnp.int32, (_BM, _LANES), 1)
    key = jnp.where(rv == mn, rc * _LANES + lane, _N_EMB)
    ind_ref[...] = jnp.min(key, axis=1, keepdims=True)
    sx = jnp.sum(x * x, axis=1, keepdims=True)
    part = 2.0 * jnp.sum(mn) + jnp.sum(sx)

    @pl.when(m == 0)
    def _():
        loss_ref[0, 0] = part

    @pl.when(m > 0)
    def _():
        loss_ref[0, 0] = loss_ref[0, 0] + part

    @pl.when(m == m_last)
    def _():
        loss_ref[0, 0] = loss_ref[0, 0] * ((1.0 + _BETA) / (_B * _DIM))


_argmin_call = pl.pallas_call(
    _argmin_body,
    grid=(_B // _BM,),
    in_specs=[
        pl.BlockSpec((_BM, _DIM), lambda m: (m, 0)),
        pl.BlockSpec((_N_EMB, _DIM), lambda m: (0, 0)),
    ],
    out_specs=[
        pl.BlockSpec((_BM, 1), lambda m: (m, 0)),
        pl.BlockSpec((1, 1), lambda m: (0, 0), memory_space=pltpu.SMEM),
    ],
    out_shape=[
        jax.ShapeDtypeStruct((_B, 1), jnp.int32),
        jax.ShapeDtypeStruct((1, 1), jnp.float32),
    ],
    scratch_shapes=[
        pltpu.VMEM((_N_EMB // _LANES, _LANES), jnp.float32),
    ],
)


def _make_gather():
    try:
        info = plsc.get_sparse_core_info()
        nc, ns = info.num_cores, info.num_subcores
    except Exception:
        nc, ns = 2, 16
    nw = nc * ns
    b_per_w = _B // nw
    ch = 128
    n_ch = b_per_w // ch
    mesh = plsc.VectorSubcoreMesh(
        core_axis_name="c", subcore_axis_name="s",
        num_cores=nc, num_subcores=ns,
    )

    @functools.partial(
        pl.kernel,
        out_type=jax.ShapeDtypeStruct((_B, _DIM), jnp.float32),
        mesh=mesh,
        scratch_types=[
            pltpu.VMEM((n_ch, ch), jnp.int32),
            pltpu.VMEM((ch, _DIM), jnp.float32),
            pltpu.VMEM((ch, _DIM), jnp.float32),
            pltpu.SemaphoreType.DMA,
            pltpu.SemaphoreType.DMA,
        ],
    )
    def gather_k(table_hbm, idx_hbm, out_hbm, idx_v, buf0, buf1, sem0, sem1):
        wid = lax.axis_index("s") * nc + lax.axis_index("c")
        base = wid * b_per_w
        pltpu.sync_copy(idx_hbm.at[wid], idx_v)
        bufs = (buf0, buf1)
        sems = (sem0, sem1)
        copies = [None, None]
        copies[0] = pltpu.async_copy(table_hbm.at[idx_v.at[0]], buf0, sem0)
        for j in range(n_ch):
            nxt = (j + 1) % 2
            if j + 1 < n_ch:
                copies[nxt] = pltpu.async_copy(
                    table_hbm.at[idx_v.at[j + 1]], bufs[nxt], sems[nxt]
                )
            copies[j % 2].wait()
            pltpu.sync_copy(
                bufs[j % 2], out_hbm.at[pl.ds(base + j * ch, ch)]
            )

    return gather_k, nw, n_ch, ch


_gather_cache = []


def kernel(x, W):
    if not _gather_cache:
        _gather_cache.append(_make_gather())
    gather_call, nw, n_ch, ch = _gather_cache[0]
    ind2, loss2 = _argmin_call(x, W)
    ind = ind2.reshape(_B)
    quantized = gather_call(W, ind.reshape(nw, n_ch, ch))
    loss = loss2[0, 0]
    return quantized, ind, loss


# speedup vs baseline: 1.1394x; 1.1394x over previous
import functools

import jax
import jax.numpy as jnp
from jax import lax
from jax.experimental import pallas as pl
from jax.experimental.pallas import tpu as pltpu
from jax.experimental.pallas import tpu_sc as plsc

_BETA = 0.25
_N_EMB = 8192
_DIM = 256
_B = 16384

_BM = 512
_BNB = 2048
_NB = _N_EMB // _BNB
_LANES = 128
_NCHB = _BNB // _LANES


def _argmin_body(x_ref, w_ref, ind_ref, loss_ref, swb_ref):
    m = pl.program_id(0)
    m_last = pl.num_programs(0) - 1

    @pl.when(m == 0)
    def _():
        w = w_ref[...]
        swb_ref[...] = (0.5 * jnp.sum(w * w, axis=1)).reshape(
            _N_EMB // _LANES, _LANES)

    x = x_ref[...]

    rv = jnp.full((_BM, _LANES), jnp.inf, jnp.float32)
    rc = jnp.zeros((_BM, _LANES), jnp.int32)
    for nb in range(_NB):
        dotm = lax.dot_general(
            x, w_ref[nb * _BNB:(nb + 1) * _BNB, :],
            (((1,), (1,)), ((), ())),
            preferred_element_type=jnp.float32,
        )
        for c in range(_NCHB):
            g = nb * _NCHB + c
            d = swb_ref[pl.ds(g, 1), :] - dotm[:, c * _LANES:(c + 1) * _LANES]
            bet = d < rv
            rv = jnp.where(bet, d, rv)
            rc = jnp.where(bet, g, rc)

    mn = jnp.min(rv, axis=1, keepdims=True)
    lane = lax.broadcasted_iota(jnp.int32, (_BM, _LANES), 1)
    key = jnp.where(rv == mn, rc * _LANES + lane, _N_EMB)
    ind_ref[...] = jnp.min(key, axis=1, keepdims=True)
    sx = jnp.sum(x * x, axis=1, keepdims=True)
    part = 2.0 * jnp.sum(mn) + jnp.sum(sx)

    @pl.when(m == 0)
    def _():
        loss_ref[0, 0] = part

    @pl.when(m > 0)
    def _():
        loss_ref[0, 0] = loss_ref[0, 0] + part

    @pl.when(m == m_last)
    def _():
        loss_ref[0, 0] = loss_ref[0, 0] * ((1.0 + _BETA) / (_B * _DIM))


_argmin_call = pl.pallas_call(
    _argmin_body,
    grid=(_B // _BM,),
    in_specs=[
        pl.BlockSpec((_BM, _DIM), lambda m: (m, 0)),
        pl.BlockSpec((_N_EMB, _DIM), lambda m: (0, 0)),
    ],
    out_specs=[
        pl.BlockSpec((_BM, 1), lambda m: (m, 0)),
        pl.BlockSpec((1, 1), lambda m: (0, 0), memory_space=pltpu.SMEM),
    ],
    out_shape=[
        jax.ShapeDtypeStruct((_B, 1), jnp.int32),
        jax.ShapeDtypeStruct((1, 1), jnp.float32),
    ],
    scratch_shapes=[
        pltpu.VMEM((_N_EMB // _LANES, _LANES), jnp.float32),
    ],
)


def _make_gather():
    try:
        info = plsc.get_sparse_core_info()
        nc, ns = info.num_cores, info.num_subcores
    except Exception:
        nc, ns = 2, 16
    nw = nc * ns
    b_per_w = _B // nw
    ch = 128
    n_ch = b_per_w // ch
    mesh = plsc.VectorSubcoreMesh(
        core_axis_name="c", subcore_axis_name="s",
        num_cores=nc, num_subcores=ns,
    )

    @functools.partial(
        pl.kernel,
        out_type=jax.ShapeDtypeStruct((_B, _DIM), jnp.float32),
        mesh=mesh,
        scratch_types=[
            pltpu.VMEM((n_ch, ch), jnp.int32),
            pltpu.VMEM((ch, _DIM), jnp.float32),
            pltpu.VMEM((ch, _DIM), jnp.float32),
            pltpu.SemaphoreType.DMA,
            pltpu.SemaphoreType.DMA,
        ],
    )
    def gather_k(table_hbm, idx_hbm, out_hbm, idx_v, buf0, buf1, sem0, sem1):
        wid = lax.axis_index("s") * nc + lax.axis_index("c")
        base = wid * b_per_w
        pltpu.sync_copy(idx_hbm.at[wid], idx_v)
        bufs = (buf0, buf1)
        sems = (sem0, sem1)
        copies = [None, None]
        copies[0] = pltpu.async_copy(table_hbm.at[idx_v.at[0]], buf0, sem0)
        for j in range(n_ch):
            nxt = (j + 1) % 2
            if j + 1 < n_ch:
                copies[nxt] = pltpu.async_copy(
                    table_hbm.at[idx_v.at[j + 1]], bufs[nxt], sems[nxt]
                )
            copies[j % 2].wait()
            pltpu.sync_copy(
                bufs[j % 2], out_hbm.at[pl.ds(base + j * ch, ch)]
            )

    return gather_k, nw, n_ch, ch


_gather_cache = []


def kernel(x, W):
    if not _gather_cache:
        _gather_cache.append(_make_gather())
    gather_call, nw, n_ch, ch = _gather_cache[0]
    ind2, loss2 = _argmin_call(x, W)
    ind = ind2.reshape(_B)
    quantized = gather_call(W, ind.reshape(nw, n_ch, ch))
    loss = loss2[0, 0]
    return quantized, ind, loss
